# Initial kernel scaffold; baseline (speedup 1.0000x reference)
import jax
import jax.numpy as jnp
from jax.experimental import pallas as pl


def kernel(mesh_pos, edges, states, node_type, pos_enc, params):
    raise NotImplementedError("write your pallas kernel here")


# speedup vs baseline: 594.1783x; 594.1783x over previous
import functools

import jax
import jax.numpy as jnp
from jax.experimental import pallas as pl
from jax.experimental.pallas import tpu as pltpu

H = 128
LN_EPS = 1e-5
BE = 3200
BN = 2000

_F32 = jnp.float32


def _dot(a, b):
    return jnp.dot(a, b, preferred_element_type=_F32)


def _layernorm(x, g, b):
    mu = jnp.mean(x, axis=-1, keepdims=True)
    var = jnp.mean((x - mu) ** 2, axis=-1, keepdims=True)
    return (x - mu) * jax.lax.rsqrt(var + LN_EPS) * g + b


def _enc_node_body(st, nt, pe, w0s, w0t, b0, w1, b1, wsv, wsp, wrv, wrp,
                   v_out, xs_out, xr_out):
    h = jax.nn.relu(_dot(st[...], w0s[...]) + _dot(nt[...], w0t[...]) + b0[...])
    v = _dot(h, w1[...]) + b1[...]
    v_out[...] = v
    pe_v = pe[...]
    xs_out[...] = _dot(v, wsv[...]) + _dot(pe_v, wsp[...])
    xr_out[...] = _dot(v, wrv[...]) + _dot(pe_v, wrp[...])


def _enc_edge_body(mps, mpr, w0d, w0n, b0, w1, b1, e_out):
    d = mps[...] - mpr[...]
    nrm = jnp.sqrt(jnp.sum(d * d, axis=-1, keepdims=True))
    pre = _dot(d, w0d[...]) + nrm * w0n[...] + b0[...]
    h = jax.nn.relu(pre)
    e_out[...] = _dot(h, w1[...]) + b1[...]


def _gn_edge_body(gs, gr, e, we, b0, w1, b1, g, bn, enew_out, eout_out):
    e_v = e[...]
    pre = gs[...] + gr[...] + _dot(e_v, we[...]) + b0[...]
    h = jax.nn.relu(pre)
    p = _dot(h, w1[...]) + b1[...]
    en = _layernorm(p, g[...], bn[...])
    enew_out[...] = en
    eout_out[...] = e_v + en


def _gn_node_body(v, pe, agg, wv, wp, wa, b0, w1, b1, g, bn,
                  wsv, wsp, wrv, wrp, *outs, last):
    v_v = v[...]
    pe_v = pe[...]
    pre = (_dot(v_v, wv[...]) + _dot(pe_v, wp[...]) + _dot(agg[...], wa[...])
           + b0[...])
    h = jax.nn.relu(pre)
    p = _dot(h, w1[...]) + b1[...]
    vn = _layernorm(p, g[...], bn[...])
    vout = v_v + vn
    outs[0][...] = vout
    if not last:
        outs[1][...] = _dot(vout, wsv[...]) + _dot(pe_v, wsp[...])
        outs[2][...] = _dot(vout, wrv[...]) + _dot(pe_v, wrp[...])


def _row_spec(d):
    return pl.BlockSpec((BN, d), lambda i: (i, 0))


def _erow_spec(d):
    return pl.BlockSpec((BE, d), lambda i: (i, 0))


def _w_spec(a, b):
    return pl.BlockSpec((a, b), lambda i: (0, 0))


def _enc_node(st, nt, pe, w0s, w0t, b0, w1, b1, wsv, wsp, wrv, wrp):
    n = st.shape[0]
    grid = (n // BN,)
    out = jax.ShapeDtypeStruct((n, H), _F32)
    return pl.pallas_call(
        _enc_node_body,
        grid=grid,
        in_specs=[_row_spec(3), _row_spec(9), _row_spec(pe.shape[1]),
                  _w_spec(3, H), _w_spec(9, H), _w_spec(1, H),
                  _w_spec(H, H), _w_spec(1, H),
                  _w_spec(H, H), _w_spec(pe.shape[1], H),
                  _w_spec(H, H), _w_spec(pe.shape[1], H)],
        out_specs=[_row_spec(H), _row_spec(H), _row_spec(H)],
        out_shape=[out, out, out],
    )(st, nt, pe, w0s, w0t, b0, w1, b1, wsv, wsp, wrv, wrp)


def _enc_edge(mps, mpr, w0d, w0n, b0, w1, b1):
    e = mps.shape[0]
    grid = (e // BE,)
    return pl.pallas_call(
        _enc_edge_body,
        grid=grid,
        in_specs=[_erow_spec(2), _erow_spec(2),
                  _w_spec(2, H), _w_spec(1, H), _w_spec(1, H),
                  _w_spec(H, H), _w_spec(1, H)],
        out_specs=_erow_spec(H),
        out_shape=jax.ShapeDtypeStruct((e, H), _F32),
    )(mps, mpr, w0d, w0n, b0, w1, b1)


def _gn_edge(gs, gr, e_in, we, b0, w1, b1, g, bn):
    e = gs.shape[0]
    grid = (e // BE,)
    out = jax.ShapeDtypeStruct((e, H), _F32)
    return pl.pallas_call(
        _gn_edge_body,
        grid=grid,
        in_specs=[_erow_spec(H), _erow_spec(H), _erow_spec(H),
                  _w_spec(H, H), _w_spec(1, H), _w_spec(H, H), _w_spec(1, H),
                  _w_spec(1, H), _w_spec(1, H)],
        out_specs=[_erow_spec(H), _erow_spec(H)],
        out_shape=[out, out],
    )(gs, gr, e_in, we, b0, w1, b1, g, bn)


def _gn_node(v, pe, agg, wv, wp, wa, b0, w1, b1, g, bn, wsv, wsp, wrv, wrp,
             last):
    n = v.shape[0]
    grid = (n // BN,)
    out = jax.ShapeDtypeStruct((n, H), _F32)
    pd = pe.shape[1]
    body = functools.partial(_gn_node_body, last=last)
    n_out = 1 if last else 3
    res = pl.pallas_call(
        body,
        grid=grid,
        in_specs=[_row_spec(H), _row_spec(pd), _row_spec(H),
                  _w_spec(H, H), _w_spec(pd, H), _w_spec(H, H), _w_spec(1, H),
                  _w_spec(H, H), _w_spec(1, H), _w_spec(1, H), _w_spec(1, H),
                  _w_spec(H, H), _w_spec(pd, H), _w_spec(H, H), _w_spec(pd, H)],
        out_specs=[_row_spec(H)] * n_out,
        out_shape=[out] * n_out,
    )(v, pe, agg, wv, wp, wa, b0, w1, b1, g, bn, wsv, wsp, wrv, wrp)
    if last:
        return res[0], None, None
    return res


def _b(x):
    return x.reshape(1, H)


def kernel(mesh_pos, edges, states, node_type, pos_enc, params):
    mp = mesh_pos[0]
    s_idx = edges[0, :, 0]
    r_idx = edges[0, :, 1]
    st = states[0]
    nt = node_type[0]
    pe = pos_enc[0]
    n = st.shape[0]
    pd = pe.shape[1]

    gns = params["gns"]

    enw = params["enc_node"]
    en_w0 = enw["l0"]["w"]
    ee = params["enc_edge"]
    ee_w0 = ee["l0"]["w"]

    def edge_l0_split(i):
        w0 = gns[i]["edge"]["l0"]["w"]
        wsv = w0[0:H]
        wsp = w0[H:H + pd]
        wrv = w0[H + pd:2 * H + pd]
        wrp = w0[2 * H + pd:2 * (H + pd)]
        we = w0[2 * (H + pd):]
        return wsv, wsp, wrv, wrp, we

    def node_l0_split(i):
        w0 = gns[i]["node"]["l0"]["w"]
        return w0[0:H], w0[H:H + pd], w0[H + pd:]

    wsv0, wsp0, wrv0, wrp0, we0 = edge_l0_split(0)
    v_cur, xs, xr = _enc_node(
        st, nt, pe,
        en_w0[0:3], en_w0[3:12], _b(enw["l0"]["b"]),
        enw["l1"]["w"], _b(enw["l1"]["b"]),
        wsv0, wsp0, wrv0, wrp0)

    mps = jnp.take(mp, s_idx, axis=0)
    mpr = jnp.take(mp, r_idx, axis=0)
    e_cur = _enc_edge(mps, mpr,
                      ee_w0[0:2], ee_w0[2:3], _b(ee["l0"]["b"]),
                      ee["l1"]["w"], _b(ee["l1"]["b"]))

    for i in range(len(gns)):
        gp = gns[i]
        last = i == len(gns) - 1
        _, _, _, _, we = edge_l0_split(i)

        gs = jnp.take(xs, s_idx, axis=0)
        gr = jnp.take(xr, r_idx, axis=0)
        e_new, e_cur = _gn_edge(
            gs, gr, e_cur, we, _b(gp["edge"]["l0"]["b"]),
            gp["edge"]["l1"]["w"], _b(gp["edge"]["l1"]["b"]),
            _b(gp["edge"]["g"]), _b(gp["edge"]["bn"]))

        agg = jnp.zeros((n, H), _F32).at[r_idx].add(e_new)

        wv, wp, wa = node_l0_split(i)
        if last:
            wsv, wsp, wrv, wrp = wsv0, wsp0, wrv0, wrp0
        else:
            wsv, wsp, wrv, wrp, _ = edge_l0_split(i + 1)
        v_cur, xs, xr = _gn_node(
            v_cur, pe, agg, wv, wp, wa, _b(gp["node"]["l0"]["b"]),
            gp["node"]["l1"]["w"], _b(gp["node"]["l1"]["b"]),
            _b(gp["node"]["g"]), _b(gp["node"]["bn"]),
            wsv, wsp, wrv, wrp, last)

    return (v_cur[None], e_cur[None])


# speedup vs baseline: 1043.9885x; 1.7570x over previous
import functools

import jax
import jax.numpy as jnp
from jax.experimental import pallas as pl
from jax.experimental.pallas import tpu as pltpu
from jax.experimental.pallas import tpu_sc as plsc

H = 128
LN_EPS = 1e-5
BE = 3200
BN = 2000

_F32 = jnp.float32


def _dot(a, b):
    return jnp.dot(a, b, preferred_element_type=_F32)


def _layernorm(x, g, b):
    mu = jnp.mean(x, axis=-1, keepdims=True)
    var = jnp.mean((x - mu) ** 2, axis=-1, keepdims=True)
    return (x - mu) * jax.lax.rsqrt(var + LN_EPS) * g + b


_IDXW = 80
_KCH = 125


def _sc_gather_body(xs_hbm, xr_hbm, si_hbm, ri_hbm, gs_hbm, gr_hbm,
                    idx_s, idx_r, buf0, buf1, buf2, buf3, sem_g, sem_o):
    cid = jax.lax.axis_index("c")
    sid = jax.lax.axis_index("s")
    wid = sid * 2 + cid
    perw = _KCH * _IDXW
    ebase = wid * perw
    pltpu.sync_copy(si_hbm.at[pl.ds(ebase, perw)], idx_s)
    pltpu.sync_copy(ri_hbm.at[pl.ds(ebase, perw)], idx_r)

    def pair(j0, j1):
        c0 = pltpu.async_copy(xs_hbm.at[idx_s.at[pl.ds(j0 * _IDXW, _IDXW)]], buf0, sem_g)
        c1 = pltpu.async_copy(xr_hbm.at[idx_r.at[pl.ds(j0 * _IDXW, _IDXW)]], buf1, sem_g)
        c2 = pltpu.async_copy(xs_hbm.at[idx_s.at[pl.ds(j1 * _IDXW, _IDXW)]], buf2, sem_g)
        c3 = pltpu.async_copy(xr_hbm.at[idx_r.at[pl.ds(j1 * _IDXW, _IDXW)]], buf3, sem_g)
        c0.wait()
        c1.wait()
        c2.wait()
        c3.wait()
        o0 = pltpu.async_copy(buf0, gs_hbm.at[pl.ds(ebase + j0 * _IDXW, _IDXW)], sem_o)
        o1 = pltpu.async_copy(buf1, gr_hbm.at[pl.ds(ebase + j0 * _IDXW, _IDXW)], sem_o)
        o2 = pltpu.async_copy(buf2, gs_hbm.at[pl.ds(ebase + j1 * _IDXW, _IDXW)], sem_o)
        o3 = pltpu.async_copy(buf3, gr_hbm.at[pl.ds(ebase + j1 * _IDXW, _IDXW)], sem_o)
        o0.wait()
        o1.wait()
        o2.wait()
        o3.wait()

    def body(g, carry):
        pair(g * 2, g * 2 + 1)
        return carry

    jax.lax.fori_loop(0, _KCH // 2, body, 0)
    if _KCH % 2:
        pair(_KCH - 1, _KCH - 1)


def _sc_gather_pair(xs, xr, si, ri):
    e = si.shape[0]
    out = jax.ShapeDtypeStruct((e, H), _F32)
    f = pl.kernel(
        _sc_gather_body,
        out_type=[out, out],
        mesh=plsc.VectorSubcoreMesh(core_axis_name="c", subcore_axis_name="s"),
        scratch_types=[
            pltpu.VMEM((_KCH * _IDXW,), jnp.int32),
            pltpu.VMEM((_KCH * _IDXW,), jnp.int32),
            pltpu.VMEM((_IDXW, H), _F32),
            pltpu.VMEM((_IDXW, H), _F32),
            pltpu.VMEM((_IDXW, H), _F32),
            pltpu.VMEM((_IDXW, H), _F32),
            pltpu.SemaphoreType.DMA,
            pltpu.SemaphoreType.DMA,
        ],
    )
    return f(xs, xr, si, ri)


def _enc_node_body(st, nt, pe, w0s, w0t, b0, w1, b1, wsv, wsp, wrv, wrp,
                   v_out, xs_out, xr_out):
    h = jax.nn.relu(_dot(st[...], w0s[...]) + _dot(nt[...], w0t[...]) + b0[...])
    v = _dot(h, w1[...]) + b1[...]
    v_out[...] = v
    pe_v = pe[...]
    xs_out[...] = _dot(v, wsv[...]) + _dot(pe_v, wsp[...])
    xr_out[...] = _dot(v, wrv[...]) + _dot(pe_v, wrp[...])


def _enc_edge_body(mps, mpr, w0d, w0n, b0, w1, b1, e_out):
    d = mps[...] - mpr[...]
    nrm = jnp.sqrt(jnp.sum(d * d, axis=-1, keepdims=True))
    pre = _dot(d, w0d[...]) + nrm * w0n[...] + b0[...]
    h = jax.nn.relu(pre)
    e_out[...] = _dot(h, w1[...]) + b1[...]


def _gn_edge_body(gs, gr, e, we, b0, w1, b1, g, bn, enew_out, eout_out):
    e_v = e[...]
    pre = gs[...] + gr[...] + _dot(e_v, we[...]) + b0[...]
    h = jax.nn.relu(pre)
    p = _dot(h, w1[...]) + b1[...]
    en = _layernorm(p, g[...], bn[...])
    enew_out[...] = en
    eout_out[...] = e_v + en


def _gn_node_body(v, pe, agg, wv, wp, wa, b0, w1, b1, g, bn,
                  wsv, wsp, wrv, wrp, *outs, last):
    v_v = v[...]
    pe_v = pe[...]
    pre = (_dot(v_v, wv[...]) + _dot(pe_v, wp[...]) + _dot(agg[...], wa[...])
           + b0[...])
    h = jax.nn.relu(pre)
    p = _dot(h, w1[...]) + b1[...]
    vn = _layernorm(p, g[...], bn[...])
    vout = v_v + vn
    outs[0][...] = vout
    if not last:
        outs[1][...] = _dot(vout, wsv[...]) + _dot(pe_v, wsp[...])
        outs[2][...] = _dot(vout, wrv[...]) + _dot(pe_v, wrp[...])


def _row_spec(d):
    return pl.BlockSpec((BN, d), lambda i: (i, 0))


def _erow_spec(d):
    return pl.BlockSpec((BE, d), lambda i: (i, 0))


def _w_spec(a, b):
    return pl.BlockSpec((a, b), lambda i: (0, 0))


def _enc_node(st, nt, pe, w0s, w0t, b0, w1, b1, wsv, wsp, wrv, wrp):
    n = st.shape[0]
    grid = (n // BN,)
    out = jax.ShapeDtypeStruct((n, H), _F32)
    return pl.pallas_call(
        _enc_node_body,
        grid=grid,
        in_specs=[_row_spec(3), _row_spec(9), _row_spec(pe.shape[1]),
                  _w_spec(3, H), _w_spec(9, H), _w_spec(1, H),
                  _w_spec(H, H), _w_spec(1, H),
                  _w_spec(H, H), _w_spec(pe.shape[1], H),
                  _w_spec(H, H), _w_spec(pe.shape[1], H)],
        out_specs=[_row_spec(H), _row_spec(H), _row_spec(H)],
        out_shape=[out, out, out],
    )(st, nt, pe, w0s, w0t, b0, w1, b1, wsv, wsp, wrv, wrp)


def _enc_edge(mps, mpr, w0d, w0n, b0, w1, b1):
    e = mps.shape[0]
    grid = (e // BE,)
    return pl.pallas_call(
        _enc_edge_body,
        grid=grid,
        in_specs=[_erow_spec(2), _erow_spec(2),
                  _w_spec(2, H), _w_spec(1, H), _w_spec(1, H),
                  _w_spec(H, H), _w_spec(1, H)],
        out_specs=_erow_spec(H),
        out_shape=jax.ShapeDtypeStruct((e, H), _F32),
    )(mps, mpr, w0d, w0n, b0, w1, b1)


def _gn_edge(gs, gr, e_in, we, b0, w1, b1, g, bn):
    e = gs.shape[0]
    grid = (e // BE,)
    out = jax.ShapeDtypeStruct((e, H), _F32)
    return pl.pallas_call(
        _gn_edge_body,
        grid=grid,
        in_specs=[_erow_spec(H), _erow_spec(H), _erow_spec(H),
                  _w_spec(H, H), _w_spec(1, H), _w_spec(H, H), _w_spec(1, H),
                  _w_spec(1, H), _w_spec(1, H)],
        out_specs=[_erow_spec(H), _erow_spec(H)],
        out_shape=[out, out],
    )(gs, gr, e_in, we, b0, w1, b1, g, bn)


def _gn_node(v, pe, agg, wv, wp, wa, b0, w1, b1, g, bn, wsv, wsp, wrv, wrp,
             last):
    n = v.shape[0]
    grid = (n // BN,)
    out = jax.ShapeDtypeStruct((n, H), _F32)
    pd = pe.shape[1]
    body = functools.partial(_gn_node_body, last=last)
    n_out = 1 if last else 3
    res = pl.pallas_call(
        body,
        grid=grid,
        in_specs=[_row_spec(H), _row_spec(pd), _row_spec(H),
                  _w_spec(H, H), _w_spec(pd, H), _w_spec(H, H), _w_spec(1, H),
                  _w_spec(H, H), _w_spec(1, H), _w_spec(1, H), _w_spec(1, H),
                  _w_spec(H, H), _w_spec(pd, H), _w_spec(H, H), _w_spec(pd, H)],
        out_specs=[_row_spec(H)] * n_out,
        out_shape=[out] * n_out,
    )(v, pe, agg, wv, wp, wa, b0, w1, b1, g, bn, wsv, wsp, wrv, wrp)
    if last:
        return res[0], None, None
    return res


def _b(x):
    return x.reshape(1, H)


def kernel(mesh_pos, edges, states, node_type, pos_enc, params):
    mp = mesh_pos[0]
    s_idx = edges[0, :, 0]
    r_idx = edges[0, :, 1]
    st = states[0]
    nt = node_type[0]
    pe = pos_enc[0]
    n = st.shape[0]
    pd = pe.shape[1]

    gns = params["gns"]

    enw = params["enc_node"]
    en_w0 = enw["l0"]["w"]
    ee = params["enc_edge"]
    ee_w0 = ee["l0"]["w"]

    def edge_l0_split(i):
        w0 = gns[i]["edge"]["l0"]["w"]
        wsv = w0[0:H]
        wsp = w0[H:H + pd]
        wrv = w0[H + pd:2 * H + pd]
        wrp = w0[2 * H + pd:2 * (H + pd)]
        we = w0[2 * (H + pd):]
        return wsv, wsp, wrv, wrp, we

    def node_l0_split(i):
        w0 = gns[i]["node"]["l0"]["w"]
        return w0[0:H], w0[H:H + pd], w0[H + pd:]

    wsv0, wsp0, wrv0, wrp0, we0 = edge_l0_split(0)
    v_cur, xs, xr = _enc_node(
        st, nt, pe,
        en_w0[0:3], en_w0[3:12], _b(enw["l0"]["b"]),
        enw["l1"]["w"], _b(enw["l1"]["b"]),
        wsv0, wsp0, wrv0, wrp0)

    mps = jnp.take(mp, s_idx, axis=0)
    mpr = jnp.take(mp, r_idx, axis=0)
    e_cur = _enc_edge(mps, mpr,
                      ee_w0[0:2], ee_w0[2:3], _b(ee["l0"]["b"]),
                      ee["l1"]["w"], _b(ee["l1"]["b"]))

    for i in range(len(gns)):
        gp = gns[i]
        last = i == len(gns) - 1
        _, _, _, _, we = edge_l0_split(i)

        gs, gr = _sc_gather_pair(xs, xr, s_idx, r_idx)
        e_new, e_cur = _gn_edge(
            gs, gr, e_cur, we, _b(gp["edge"]["l0"]["b"]),
            gp["edge"]["l1"]["w"], _b(gp["edge"]["l1"]["b"]),
            _b(gp["edge"]["g"]), _b(gp["edge"]["bn"]))

        agg = jnp.zeros((n, H), _F32).at[r_idx].add(e_new)

        wv, wp, wa = node_l0_split(i)
        if last:
            wsv, wsp, wrv, wrp = wsv0, wsp0, wrv0, wrp0
        else:
            wsv, wsp, wrv, wrp, _ = edge_l0_split(i + 1)
        v_cur, xs, xr = _gn_node(
            v_cur, pe, agg, wv, wp, wa, _b(gp["node"]["l0"]["b"]),
            gp["node"]["l1"]["w"], _b(gp["node"]["l1"]["b"]),
            _b(gp["node"]["g"]), _b(gp["node"]["bn"]),
            wsv, wsp, wrv, wrp, last)

    return (v_cur[None], e_cur[None])


# speedup vs baseline: 1639.5226x; 1.5704x over previous
import functools

import jax
import jax.numpy as jnp
from jax.experimental import pallas as pl
from jax.experimental.pallas import tpu as pltpu
from jax.experimental.pallas import tpu_sc as plsc

H = 128
LN_EPS = 1e-5
BE = 3200
BN = 2000

_F32 = jnp.float32


def _dot(a, b):
    return jnp.dot(a, b, preferred_element_type=_F32)


def _layernorm(x, g, b):
    mu = jnp.mean(x, axis=-1, keepdims=True)
    var = jnp.mean((x - mu) ** 2, axis=-1, keepdims=True)
    return (x - mu) * jax.lax.rsqrt(var + LN_EPS) * g + b


_IDXW = 80
_KCH = 125


def _sc_gather_body(xs_hbm, xr_hbm, si_hbm, ri_hbm, gs_hbm, gr_hbm,
                    idx_s, idx_r, buf0, buf1, buf2, buf3, sem_g, sem_o):
    cid = jax.lax.axis_index("c")
    sid = jax.lax.axis_index("s")
    wid = sid * 2 + cid
    perw = _KCH * _IDXW
    ebase = wid * perw
    pltpu.sync_copy(si_hbm.at[pl.ds(ebase, perw)], idx_s)
    pltpu.sync_copy(ri_hbm.at[pl.ds(ebase, perw)], idx_r)

    def pair(j0, j1):
        c0 = pltpu.async_copy(xs_hbm.at[idx_s.at[pl.ds(j0 * _IDXW, _IDXW)]], buf0, sem_g)
        c1 = pltpu.async_copy(xr_hbm.at[idx_r.at[pl.ds(j0 * _IDXW, _IDXW)]], buf1, sem_g)
        c2 = pltpu.async_copy(xs_hbm.at[idx_s.at[pl.ds(j1 * _IDXW, _IDXW)]], buf2, sem_g)
        c3 = pltpu.async_copy(xr_hbm.at[idx_r.at[pl.ds(j1 * _IDXW, _IDXW)]], buf3, sem_g)
        c0.wait()
        c1.wait()
        c2.wait()
        c3.wait()
        o0 = pltpu.async_copy(buf0, gs_hbm.at[pl.ds(ebase + j0 * _IDXW, _IDXW)], sem_o)
        o1 = pltpu.async_copy(buf1, gr_hbm.at[pl.ds(ebase + j0 * _IDXW, _IDXW)], sem_o)
        o2 = pltpu.async_copy(buf2, gs_hbm.at[pl.ds(ebase + j1 * _IDXW, _IDXW)], sem_o)
        o3 = pltpu.async_copy(buf3, gr_hbm.at[pl.ds(ebase + j1 * _IDXW, _IDXW)], sem_o)
        o0.wait()
        o1.wait()
        o2.wait()
        o3.wait()

    def body(g, carry):
        pair(g * 2, g * 2 + 1)
        return carry

    jax.lax.fori_loop(0, _KCH // 2, body, 0)
    if _KCH % 2:
        pair(_KCH - 1, _KCH - 1)


def _sc_gather_pair(xs, xr, si, ri):
    e = si.shape[0]
    out = jax.ShapeDtypeStruct((e, H), _F32)
    f = pl.kernel(
        _sc_gather_body,
        out_type=[out, out],
        mesh=plsc.VectorSubcoreMesh(core_axis_name="c", subcore_axis_name="s"),
        scratch_types=[
            pltpu.VMEM((_KCH * _IDXW,), jnp.int32),
            pltpu.VMEM((_KCH * _IDXW,), jnp.int32),
            pltpu.VMEM((_IDXW, H), _F32),
            pltpu.VMEM((_IDXW, H), _F32),
            pltpu.VMEM((_IDXW, H), _F32),
            pltpu.VMEM((_IDXW, H), _F32),
            pltpu.SemaphoreType.DMA,
            pltpu.SemaphoreType.DMA,
        ],
    )
    return f(xs, xr, si, ri)


_ZROWS = 632


def _sc_scatter_body(enew_hbm, ri3d_hbm, zeros_hbm, out_hbm,
                     idx_v, rows0, rows1, agg_sh, sem_i):
    cid = jax.lax.axis_index("c")
    sid = jax.lax.axis_index("s")
    wid = sid * 2 + cid
    n = agg_sh.shape[0]
    perw = _KCH * _IDXW
    base = wid * perw

    zoff = sid * _ZROWS
    tail = n - 15 * _ZROWS

    @pl.when(sid < 15)
    def _():
        pltpu.sync_copy(zeros_hbm, agg_sh.at[pl.ds(zoff, _ZROWS)])

    @pl.when(sid == 15)
    def _():
        pltpu.sync_copy(zeros_hbm.at[pl.ds(0, tail)],
                        agg_sh.at[pl.ds(15 * _ZROWS, tail)])

    pltpu.sync_copy(ri3d_hbm.at[wid], idx_v)
    plsc.subcore_barrier()

    def pair(j0, j1):
        c0 = pltpu.async_copy(enew_hbm.at[pl.ds(base + j0 * _IDXW, _IDXW)],
                              rows0, sem_i)
        c1 = pltpu.async_copy(enew_hbm.at[pl.ds(base + j1 * _IDXW, _IDXW)],
                              rows1, sem_i)
        c0.wait()
        pltpu.sync_copy(rows0, agg_sh.at[idx_v.at[j0]], add=True)
        c1.wait()
        pltpu.sync_copy(rows1, agg_sh.at[idx_v.at[j1]], add=True)

    def body(g, carry):
        pair(g * 2, g * 2 + 1)
        return carry

    jax.lax.fori_loop(0, _KCH // 2, body, 0)
    if _KCH % 2:
        c0 = pltpu.async_copy(
            enew_hbm.at[pl.ds(base + (_KCH - 1) * _IDXW, _IDXW)], rows0, sem_i)
        c0.wait()
        pltpu.sync_copy(rows0, agg_sh.at[idx_v.at[_KCH - 1]], add=True)

    plsc.subcore_barrier()

    @pl.when(sid < 15)
    def _():
        pltpu.sync_copy(agg_sh.at[pl.ds(zoff, _ZROWS)],
                        out_hbm.at[cid, pl.ds(zoff, _ZROWS)])

    @pl.when(sid == 15)
    def _():
        pltpu.sync_copy(agg_sh.at[pl.ds(15 * _ZROWS, tail)],
                        out_hbm.at[cid, pl.ds(15 * _ZROWS, tail)])


def _sc_scatter(e_new, ri3d, zeros, n):
    f = pl.kernel(
        _sc_scatter_body,
        out_type=jax.ShapeDtypeStruct((2, n, H), _F32),
        mesh=plsc.VectorSubcoreMesh(core_axis_name="c", subcore_axis_name="s"),
        scratch_types=[
            pltpu.VMEM((_KCH, _IDXW), jnp.int32),
            pltpu.VMEM((_IDXW, H), _F32),
            pltpu.VMEM((_IDXW, H), _F32),
            pltpu.VMEM_SHARED((n, H), _F32),
            pltpu.SemaphoreType.DMA,
        ],
    )
    return f(e_new, ri3d, zeros)


def _enc_node_body(st, nt, pe, w0s, w0t, b0, w1, b1, wsv, wsp, wrv, wrp,
                   v_out, xs_out, xr_out):
    h = jax.nn.relu(_dot(st[...], w0s[...]) + _dot(nt[...], w0t[...]) + b0[...])
    v = _dot(h, w1[...]) + b1[...]
    v_out[...] = v
    pe_v = pe[...]
    xs_out[...] = _dot(v, wsv[...]) + _dot(pe_v, wsp[...])
    xr_out[...] = _dot(v, wrv[...]) + _dot(pe_v, wrp[...])


def _enc_edge_body(mps, mpr, w0d, w0n, b0, w1, b1, e_out):
    d = mps[...] - mpr[...]
    nrm = jnp.sqrt(jnp.sum(d * d, axis=-1, keepdims=True))
    pre = _dot(d, w0d[...]) + nrm * w0n[...] + b0[...]
    h = jax.nn.relu(pre)
    e_out[...] = _dot(h, w1[...]) + b1[...]


def _gn_edge_body(gs, gr, e, we, b0, w1, b1, g, bn, enew_out, eout_out):
    e_v = e[...]
    pre = gs[...] + gr[...] + _dot(e_v, we[...]) + b0[...]
    h = jax.nn.relu(pre)
    p = _dot(h, w1[...]) + b1[...]
    en = _layernorm(p, g[...], bn[...])
    enew_out[...] = en
    eout_out[...] = e_v + en


def _gn_node_body(v, pe, a0, a1, wv, wp, wa, b0, w1, b1, g, bn,
                  wsv, wsp, wrv, wrp, *outs, last):
    v_v = v[...]
    pe_v = pe[...]
    pre = (_dot(v_v, wv[...]) + _dot(pe_v, wp[...])
           + _dot(a0[...] + a1[...], wa[...]) + b0[...])
    h = jax.nn.relu(pre)
    p = _dot(h, w1[...]) + b1[...]
    vn = _layernorm(p, g[...], bn[...])
    vout = v_v + vn
    outs[0][...] = vout
    if not last:
        outs[1][...] = _dot(vout, wsv[...]) + _dot(pe_v, wsp[...])
        outs[2][...] = _dot(vout, wrv[...]) + _dot(pe_v, wrp[...])


def _row_spec(d):
    return pl.BlockSpec((BN, d), lambda i: (i, 0))


def _erow_spec(d):
    return pl.BlockSpec((BE, d), lambda i: (i, 0))


def _w_spec(a, b):
    return pl.BlockSpec((a, b), lambda i: (0, 0))


def _enc_node(st, nt, pe, w0s, w0t, b0, w1, b1, wsv, wsp, wrv, wrp):
    n = st.shape[0]
    grid = (n // BN,)
    out = jax.ShapeDtypeStruct((n, H), _F32)
    return pl.pallas_call(
        _enc_node_body,
        grid=grid,
        in_specs=[_row_spec(3), _row_spec(9), _row_spec(pe.shape[1]),
                  _w_spec(3, H), _w_spec(9, H), _w_spec(1, H),
                  _w_spec(H, H), _w_spec(1, H),
                  _w_spec(H, H), _w_spec(pe.shape[1], H),
                  _w_spec(H, H), _w_spec(pe.shape[1], H)],
        out_specs=[_row_spec(H), _row_spec(H), _row_spec(H)],
        out_shape=[out, out, out],
    )(st, nt, pe, w0s, w0t, b0, w1, b1, wsv, wsp, wrv, wrp)


def _enc_edge(mps, mpr, w0d, w0n, b0, w1, b1):
    e = mps.shape[0]
    grid = (e // BE,)
    return pl.pallas_call(
        _enc_edge_body,
        grid=grid,
        in_specs=[_erow_spec(2), _erow_spec(2),
                  _w_spec(2, H), _w_spec(1, H), _w_spec(1, H),
                  _w_spec(H, H), _w_spec(1, H)],
        out_specs=_erow_spec(H),
        out_shape=jax.ShapeDtypeStruct((e, H), _F32),
    )(mps, mpr, w0d, w0n, b0, w1, b1)


def _gn_edge(gs, gr, e_in, we, b0, w1, b1, g, bn):
    e = gs.shape[0]
    grid = (e // BE,)
    out = jax.ShapeDtypeStruct((e, H), _F32)
    return pl.pallas_call(
        _gn_edge_body,
        grid=grid,
        in_specs=[_erow_spec(H), _erow_spec(H), _erow_spec(H),
                  _w_spec(H, H), _w_spec(1, H), _w_spec(H, H), _w_spec(1, H),
                  _w_spec(1, H), _w_spec(1, H)],
        out_specs=[_erow_spec(H), _erow_spec(H)],
        out_shape=[out, out],
    )(gs, gr, e_in, we, b0, w1, b1, g, bn)


def _gn_node(v, pe, a0, a1, wv, wp, wa, b0, w1, b1, g, bn, wsv, wsp, wrv, wrp,
             last):
    n = v.shape[0]
    grid = (n // BN,)
    out = jax.ShapeDtypeStruct((n, H), _F32)
    pd = pe.shape[1]
    body = functools.partial(_gn_node_body, last=last)
    n_out = 1 if last else 3
    res = pl.pallas_call(
        body,
        grid=grid,
        in_specs=[_row_spec(H), _row_spec(pd), _row_spec(H), _row_spec(H),
                  _w_spec(H, H), _w_spec(pd, H), _w_spec(H, H), _w_spec(1, H),
                  _w_spec(H, H), _w_spec(1, H), _w_spec(1, H), _w_spec(1, H),
                  _w_spec(H, H), _w_spec(pd, H), _w_spec(H, H), _w_spec(pd, H)],
        out_specs=[_row_spec(H)] * n_out,
        out_shape=[out] * n_out,
    )(v, pe, a0, a1, wv, wp, wa, b0, w1, b1, g, bn, wsv, wsp, wrv, wrp)
    if last:
        return res[0], None, None
    return res


def _b(x):
    return x.reshape(1, H)


def kernel(mesh_pos, edges, states, node_type, pos_enc, params):
    mp = mesh_pos[0]
    s_idx = edges[0, :, 0]
    r_idx = edges[0, :, 1]
    st = states[0]
    nt = node_type[0]
    pe = pos_enc[0]
    n = st.shape[0]
    pd = pe.shape[1]
    ri3d = r_idx.reshape(32, _KCH, _IDXW)
    zrows = jnp.zeros((_ZROWS, H), _F32)

    gns = params["gns"]

    enw = params["enc_node"]
    en_w0 = enw["l0"]["w"]
    ee = params["enc_edge"]
    ee_w0 = ee["l0"]["w"]

    def edge_l0_split(i):
        w0 = gns[i]["edge"]["l0"]["w"]
        wsv = w0[0:H]
        wsp = w0[H:H + pd]
        wrv = w0[H + pd:2 * H + pd]
        wrp = w0[2 * H + pd:2 * (H + pd)]
        we = w0[2 * (H + pd):]
        return wsv, wsp, wrv, wrp, we

    def node_l0_split(i):
        w0 = gns[i]["node"]["l0"]["w"]
        return w0[0:H], w0[H:H + pd], w0[H + pd:]

    wsv0, wsp0, wrv0, wrp0, we0 = edge_l0_split(0)
    v_cur, xs, xr = _enc_node(
        st, nt, pe,
        en_w0[0:3], en_w0[3:12], _b(enw["l0"]["b"]),
        enw["l1"]["w"], _b(enw["l1"]["b"]),
        wsv0, wsp0, wrv0, wrp0)

    mps = jnp.take(mp, s_idx, axis=0)
    mpr = jnp.take(mp, r_idx, axis=0)
    e_cur = _enc_edge(mps, mpr,
                      ee_w0[0:2], ee_w0[2:3], _b(ee["l0"]["b"]),
                      ee["l1"]["w"], _b(ee["l1"]["b"]))

    for i in range(len(gns)):
        gp = gns[i]
        last = i == len(gns) - 1
        _, _, _, _, we = edge_l0_split(i)

        gs, gr = _sc_gather_pair(xs, xr, s_idx, r_idx)
        e_new, e_cur = _gn_edge(
            gs, gr, e_cur, we, _b(gp["edge"]["l0"]["b"]),
            gp["edge"]["l1"]["w"], _b(gp["edge"]["l1"]["b"]),
            _b(gp["edge"]["g"]), _b(gp["edge"]["bn"]))

        agg2 = _sc_scatter(e_new, ri3d, zrows, n)

        wv, wp, wa = node_l0_split(i)
        if last:
            wsv, wsp, wrv, wrp = wsv0, wsp0, wrv0, wrp0
        else:
            wsv, wsp, wrv, wrp, _ = edge_l0_split(i + 1)
        v_cur, xs, xr = _gn_node(
            v_cur, pe, agg2[0], agg2[1], wv, wp, wa, _b(gp["node"]["l0"]["b"]),
            gp["node"]["l1"]["w"], _b(gp["node"]["l1"]["b"]),
            _b(gp["node"]["g"]), _b(gp["node"]["bn"]),
            wsv, wsp, wrv, wrp, last)

    return (v_cur[None], e_cur[None])


# speedup vs baseline: 1693.2431x; 1.0328x over previous
import functools

import jax
import jax.numpy as jnp
from jax.experimental import pallas as pl
from jax.experimental.pallas import tpu as pltpu
from jax.experimental.pallas import tpu_sc as plsc

H = 128
LN_EPS = 1e-5
BE = 3200
BN = 2000

_F32 = jnp.float32
_BF16 = jnp.bfloat16


def _dot(a, b):
    return jnp.dot(a, b, preferred_element_type=_F32)


def _pack_bf16(x):
    u = jax.lax.bitcast_convert_type(x.astype(_BF16).astype(_F32), jnp.uint32)
    return (u[:, :64] >> 16) | (u[:, 64:] & jnp.uint32(0xFFFF0000))


def _unpack_bf16(p):
    lo = jax.lax.bitcast_convert_type(p << 16, _F32)
    hi = jax.lax.bitcast_convert_type(p & jnp.uint32(0xFFFF0000), _F32)
    return jnp.concatenate([lo, hi], axis=-1)


def _layernorm(x, g, b):
    mu = jnp.mean(x, axis=-1, keepdims=True)
    var = jnp.mean((x - mu) ** 2, axis=-1, keepdims=True)
    return (x - mu) * jax.lax.rsqrt(var + LN_EPS) * g + b


_IDXW = 80
_KCH = 125


def _sc_gather_body(xs_hbm, xr_hbm, si_hbm, ri_hbm, gs_hbm, gr_hbm,
                    idx_s, idx_r, buf0, buf1, buf2, buf3, sem_g, sem_o):
    cid = jax.lax.axis_index("c")
    sid = jax.lax.axis_index("s")
    wid = sid * 2 + cid
    perw = _KCH * _IDXW
    ebase = wid * perw
    pltpu.sync_copy(si_hbm.at[pl.ds(ebase, perw)], idx_s)
    pltpu.sync_copy(ri_hbm.at[pl.ds(ebase, perw)], idx_r)

    def pair(j0, j1):
        c0 = pltpu.async_copy(xs_hbm.at[idx_s.at[pl.ds(j0 * _IDXW, _IDXW)]], buf0, sem_g)
        c1 = pltpu.async_copy(xr_hbm.at[idx_r.at[pl.ds(j0 * _IDXW, _IDXW)]], buf1, sem_g)
        c2 = pltpu.async_copy(xs_hbm.at[idx_s.at[pl.ds(j1 * _IDXW, _IDXW)]], buf2, sem_g)
        c3 = pltpu.async_copy(xr_hbm.at[idx_r.at[pl.ds(j1 * _IDXW, _IDXW)]], buf3, sem_g)
        c0.wait()
        c1.wait()
        c2.wait()
        c3.wait()
        o0 = pltpu.async_copy(buf0, gs_hbm.at[pl.ds(ebase + j0 * _IDXW, _IDXW)], sem_o)
        o1 = pltpu.async_copy(buf1, gr_hbm.at[pl.ds(ebase + j0 * _IDXW, _IDXW)], sem_o)
        o2 = pltpu.async_copy(buf2, gs_hbm.at[pl.ds(ebase + j1 * _IDXW, _IDXW)], sem_o)
        o3 = pltpu.async_copy(buf3, gr_hbm.at[pl.ds(ebase + j1 * _IDXW, _IDXW)], sem_o)
        o0.wait()
        o1.wait()
        o2.wait()
        o3.wait()

    def body(g, carry):
        pair(g * 2, g * 2 + 1)
        return carry

    jax.lax.fori_loop(0, _KCH // 2, body, 0)
    if _KCH % 2:
        pair(_KCH - 1, _KCH - 1)


def _sc_gather_pair(xs, xr, si, ri):
    e = si.shape[0]
    dt = xs.dtype
    w = xs.shape[1]
    out = jax.ShapeDtypeStruct((e, w), dt)
    f = pl.kernel(
        _sc_gather_body,
        out_type=[out, out],
        mesh=plsc.VectorSubcoreMesh(core_axis_name="c", subcore_axis_name="s"),
        scratch_types=[
            pltpu.VMEM((_KCH * _IDXW,), jnp.int32),
            pltpu.VMEM((_KCH * _IDXW,), jnp.int32),
            pltpu.VMEM((_IDXW, w), dt),
            pltpu.VMEM((_IDXW, w), dt),
            pltpu.VMEM((_IDXW, w), dt),
            pltpu.VMEM((_IDXW, w), dt),
            pltpu.SemaphoreType.DMA,
            pltpu.SemaphoreType.DMA,
        ],
    )
    return f(xs, xr, si, ri)


_ZROWS = 632


def _sc_scatter_body(enew_hbm, ri3d_hbm, zeros_hbm, out_hbm,
                     idx_v, rows0, rows1, agg_sh, sem_i):
    cid = jax.lax.axis_index("c")
    sid = jax.lax.axis_index("s")
    wid = sid * 2 + cid
    n = agg_sh.shape[0]
    perw = _KCH * _IDXW
    base = wid * perw

    zoff = sid * _ZROWS
    tail = n - 15 * _ZROWS

    @pl.when(sid < 15)
    def _():
        pltpu.sync_copy(zeros_hbm, agg_sh.at[pl.ds(zoff, _ZROWS)])

    @pl.when(sid == 15)
    def _():
        pltpu.sync_copy(zeros_hbm.at[pl.ds(0, tail)],
                        agg_sh.at[pl.ds(15 * _ZROWS, tail)])

    pltpu.sync_copy(ri3d_hbm.at[wid], idx_v)
    plsc.subcore_barrier()

    def pair(j0, j1):
        c0 = pltpu.async_copy(enew_hbm.at[pl.ds(base + j0 * _IDXW, _IDXW)],
                              rows0, sem_i)
        c1 = pltpu.async_copy(enew_hbm.at[pl.ds(base + j1 * _IDXW, _IDXW)],
                              rows1, sem_i)
        c0.wait()
        pltpu.sync_copy(rows0, agg_sh.at[idx_v.at[j0]], add=True)
        c1.wait()
        pltpu.sync_copy(rows1, agg_sh.at[idx_v.at[j1]], add=True)

    def body(g, carry):
        pair(g * 2, g * 2 + 1)
        return carry

    jax.lax.fori_loop(0, _KCH // 2, body, 0)
    if _KCH % 2:
        c0 = pltpu.async_copy(
            enew_hbm.at[pl.ds(base + (_KCH - 1) * _IDXW, _IDXW)], rows0, sem_i)
        c0.wait()
        pltpu.sync_copy(rows0, agg_sh.at[idx_v.at[_KCH - 1]], add=True)

    plsc.subcore_barrier()

    @pl.when(sid < 15)
    def _():
        pltpu.sync_copy(agg_sh.at[pl.ds(zoff, _ZROWS)],
                        out_hbm.at[cid, pl.ds(zoff, _ZROWS)])

    @pl.when(sid == 15)
    def _():
        pltpu.sync_copy(agg_sh.at[pl.ds(15 * _ZROWS, tail)],
                        out_hbm.at[cid, pl.ds(15 * _ZROWS, tail)])


def _sc_scatter(e_new, ri3d, zeros, n):
    f = pl.kernel(
        _sc_scatter_body,
        out_type=jax.ShapeDtypeStruct((2, n, H), _F32),
        mesh=plsc.VectorSubcoreMesh(core_axis_name="c", subcore_axis_name="s"),
        scratch_types=[
            pltpu.VMEM((_KCH, _IDXW), jnp.int32),
            pltpu.VMEM((_IDXW, H), _F32),
            pltpu.VMEM((_IDXW, H), _F32),
            pltpu.VMEM_SHARED((n, H), _F32),
            pltpu.SemaphoreType.DMA,
        ],
    )
    return f(e_new, ri3d, zeros)


def _enc_node_body(st, nt, pe, w0s, w0t, b0, w1, b1, wsv, wsp, wrv, wrp,
                   v_out, xs_out, xr_out):
    h = jax.nn.relu(_dot(st[...], w0s[...]) + _dot(nt[...], w0t[...]) + b0[...])
    v = _dot(h, w1[...]) + b1[...]
    v_out[...] = v
    pe_v = pe[...]
    xs_out[...] = _dot(v, wsv[...]) + _dot(pe_v, wsp[...])
    xr_out[...] = _dot(v, wrv[...]) + _dot(pe_v, wrp[...])


def _enc_edge_body(mps, mpr, w0d, w0n, b0, w1, b1, e_out):
    d = mps[...] - mpr[...]
    nrm = jnp.sqrt(jnp.sum(d * d, axis=-1, keepdims=True))
    pre = _dot(d, w0d[...]) + nrm * w0n[...] + b0[...]
    h = jax.nn.relu(pre)
    e_out[...] = (_dot(h, w1[...]) + b1[...]).astype(e_out.dtype)


def _gn_edge_body(gs, gr, e, we, b0, w1, b1, g, bn, enew_out, eout_out):
    e_v = e[...].astype(_F32)
    pre = gs[...] + gr[...] + _dot(e_v, we[...]) + b0[...]
    h = jax.nn.relu(pre)
    p = _dot(h, w1[...]) + b1[...]
    en = _layernorm(p, g[...], bn[...])
    enew_out[...] = en
    eout_out[...] = (e_v + en).astype(eout_out.dtype)


def _gn_node_body(v, pe, a0, a1, wv, wp, wa, b0, w1, b1, g, bn,
                  wsv, wsp, wrv, wrp, *outs, last):
    v_v = v[...]
    pe_v = pe[...]
    pre = (_dot(v_v, wv[...]) + _dot(pe_v, wp[...])
           + _dot(a0[...] + a1[...], wa[...]) + b0[...])
    h = jax.nn.relu(pre)
    p = _dot(h, w1[...]) + b1[...]
    vn = _layernorm(p, g[...], bn[...])
    vout = v_v + vn
    outs[0][...] = vout
    if not last:
        outs[1][...] = _dot(vout, wsv[...]) + _dot(pe_v, wsp[...])
        outs[2][...] = _dot(vout, wrv[...]) + _dot(pe_v, wrp[...])


def _row_spec(d):
    return pl.BlockSpec((BN, d), lambda i: (i, 0))


def _erow_spec(d):
    return pl.BlockSpec((BE, d), lambda i: (i, 0))


def _w_spec(a, b):
    return pl.BlockSpec((a, b), lambda i: (0, 0))


def _enc_node(st, nt, pe, w0s, w0t, b0, w1, b1, wsv, wsp, wrv, wrp):
    n = st.shape[0]
    grid = (n // BN,)
    out = jax.ShapeDtypeStruct((n, H), _F32)
    return pl.pallas_call(
        _enc_node_body,
        grid=grid,
        in_specs=[_row_spec(3), _row_spec(9), _row_spec(pe.shape[1]),
                  _w_spec(3, H), _w_spec(9, H), _w_spec(1, H),
                  _w_spec(H, H), _w_spec(1, H),
                  _w_spec(H, H), _w_spec(pe.shape[1], H),
                  _w_spec(H, H), _w_spec(pe.shape[1], H)],
        out_specs=[_row_spec(H), _row_spec(H), _row_spec(H)],
        out_shape=[out, out, out],
    )(st, nt, pe, w0s, w0t, b0, w1, b1, wsv, wsp, wrv, wrp)


def _enc_edge(mps, mpr, w0d, w0n, b0, w1, b1):
    e = mps.shape[0]
    grid = (e // BE,)
    return pl.pallas_call(
        _enc_edge_body,
        grid=grid,
        in_specs=[_erow_spec(2), _erow_spec(2),
                  _w_spec(2, H), _w_spec(1, H), _w_spec(1, H),
                  _w_spec(H, H), _w_spec(1, H)],
        out_specs=_erow_spec(H),
        out_shape=jax.ShapeDtypeStruct((e, H), _BF16),
    )(mps, mpr, w0d, w0n, b0, w1, b1)


def _gn_edge(gs, gr, e_in, we, b0, w1, b1, g, bn, eout_dtype):
    e = gs.shape[0]
    grid = (e // BE,)
    return pl.pallas_call(
        _gn_edge_body,
        grid=grid,
        in_specs=[_erow_spec(H), _erow_spec(H), _erow_spec(H),
                  _w_spec(H, H), _w_spec(1, H), _w_spec(H, H), _w_spec(1, H),
                  _w_spec(1, H), _w_spec(1, H)],
        out_specs=[_erow_spec(H), _erow_spec(H)],
        out_shape=[jax.ShapeDtypeStruct((e, H), _F32),
                   jax.ShapeDtypeStruct((e, H), eout_dtype)],
    )(gs, gr, e_in, we, b0, w1, b1, g, bn)


def _gn_node(v, pe, a0, a1, wv, wp, wa, b0, w1, b1, g, bn, wsv, wsp, wrv, wrp,
             last):
    n = v.shape[0]
    grid = (n // BN,)
    out = jax.ShapeDtypeStruct((n, H), _F32)
    pd = pe.shape[1]
    body = functools.partial(_gn_node_body, last=last)
    n_out = 1 if last else 3
    res = pl.pallas_call(
        body,
        grid=grid,
        in_specs=[_row_spec(H), _row_spec(pd), _row_spec(H), _row_spec(H),
                  _w_spec(H, H), _w_spec(pd, H), _w_spec(H, H), _w_spec(1, H),
                  _w_spec(H, H), _w_spec(1, H), _w_spec(1, H), _w_spec(1, H),
                  _w_spec(H, H), _w_spec(pd, H), _w_spec(H, H), _w_spec(pd, H)],
        out_specs=[_row_spec(H)] * n_out,
        out_shape=[out] * n_out,
    )(v, pe, a0, a1, wv, wp, wa, b0, w1, b1, g, bn, wsv, wsp, wrv, wrp)
    if last:
        return res[0], None, None
    return res


def _b(x):
    return x.reshape(1, H)


def kernel(mesh_pos, edges, states, node_type, pos_enc, params):
    mp = mesh_pos[0]
    s_idx = edges[0, :, 0]
    r_idx = edges[0, :, 1]
    st = states[0]
    nt = node_type[0]
    pe = pos_enc[0]
    n = st.shape[0]
    pd = pe.shape[1]
    ri3d = r_idx.reshape(32, _KCH, _IDXW)
    zrows = jnp.zeros((_ZROWS, H), _F32)

    gns = params["gns"]

    enw = params["enc_node"]
    en_w0 = enw["l0"]["w"]
    ee = params["enc_edge"]
    ee_w0 = ee["l0"]["w"]

    def edge_l0_split(i):
        w0 = gns[i]["edge"]["l0"]["w"]
        wsv = w0[0:H]
        wsp = w0[H:H + pd]
        wrv = w0[H + pd:2 * H + pd]
        wrp = w0[2 * H + pd:2 * (H + pd)]
        we = w0[2 * (H + pd):]
        return wsv, wsp, wrv, wrp, we

    def node_l0_split(i):
        w0 = gns[i]["node"]["l0"]["w"]
        return w0[0:H], w0[H:H + pd], w0[H + pd:]

    wsv0, wsp0, wrv0, wrp0, we0 = edge_l0_split(0)
    v_cur, xs, xr = _enc_node(
        st, nt, pe,
        en_w0[0:3], en_w0[3:12], _b(enw["l0"]["b"]),
        enw["l1"]["w"], _b(enw["l1"]["b"]),
        wsv0, wsp0, wrv0, wrp0)

    mps = jnp.take(mp, s_idx, axis=0)
    mpr = jnp.take(mp, r_idx, axis=0)
    e_cur = _enc_edge(mps, mpr,
                      ee_w0[0:2], ee_w0[2:3], _b(ee["l0"]["b"]),
                      ee["l1"]["w"], _b(ee["l1"]["b"]))

    for i in range(len(gns)):
        gp = gns[i]
        last = i == len(gns) - 1
        _, _, _, _, we = edge_l0_split(i)

        gs, gr = _sc_gather_pair(xs, xr, s_idx, r_idx)
        e_new, e_cur = _gn_edge(
            gs, gr, e_cur, we, _b(gp["edge"]["l0"]["b"]),
            gp["edge"]["l1"]["w"], _b(gp["edge"]["l1"]["b"]),
            _b(gp["edge"]["g"]), _b(gp["edge"]["bn"]),
            _F32 if last else _BF16)

        agg2 = _sc_scatter(e_new, ri3d, zrows, n)

        wv, wp, wa = node_l0_split(i)
        if last:
            wsv, wsp, wrv, wrp = wsv0, wsp0, wrv0, wrp0
        else:
            wsv, wsp, wrv, wrp, _ = edge_l0_split(i + 1)
        v_cur, xs, xr = _gn_node(
            v_cur, pe, agg2[0], agg2[1], wv, wp, wa, _b(gp["node"]["l0"]["b"]),
            gp["node"]["l1"]["w"], _b(gp["node"]["l1"]["b"]),
            _b(gp["node"]["g"]), _b(gp["node"]["bn"]),
            wsv, wsp, wrv, wrp, last)

    return (v_cur[None], e_cur[None])
